# baseline (speedup 1.0000x reference)
import functools

import jax
import jax.numpy as jnp
from jax import lax
from jax.experimental import pallas as pl
from jax.experimental.pallas import tpu as pltpu
from jax.experimental.pallas import tpu_sc as plsc

_N = 10000
_NPAD = 10240
_E = 320000
_B = 128
_NC = 2
_NS = 16
_BT = 160
_CH = 16
_NCH = _BT // _CH
_KR = 8
_EPAD = _NS * _BT * _B
_IROWS = _NS * _BT + _CH
_RPT = _NPAD // _NS
_BN_EPS = 1e-5


@functools.cache
def _get_sc_agg(npairs=1):
    mesh = plsc.VectorSubcoreMesh(
        core_axis_name="c", subcore_axis_name="s", num_cores=_NC, num_subcores=_NS
    )

    @functools.partial(
        pl.kernel,
        out_type=jax.ShapeDtypeStruct((2 * npairs, _NPAD, 64), jnp.float32),
        mesh=mesh,
        compiler_params=pltpu.CompilerParams(use_tc_tiling_on_sc=False),
        scratch_types=[
            pltpu.VMEM((2, _CH, _B), jnp.int32),
            pltpu.VMEM((2, _CH, _B), jnp.int32),
            pltpu.VMEM((_KR, _B, 64), jnp.float32),
            pltpu.VMEM_SHARED((_NPAD, 64), jnp.float32),
            [pltpu.SemaphoreType.DMA] * _KR,
            [pltpu.SemaphoreType.DMA] * _KR,
            pltpu.SemaphoreType.DMA,
        ],
    )
    def _sc_agg(*refs):
        tables = refs[: 2 * npairs]
        src2, dst2, out = refs[2 * npairs: 2 * npairs + 3]
        srcb, dstb, rows, acc, sem_g, sem_s, sem_i = refs[2 * npairs + 3:]
        c = lax.axis_index("c")
        s = lax.axis_index("s")
        r0 = s * _RPT
        row0 = s * _BT

        def wait_rows(slot, sem):
            pltpu.make_async_copy(tables[0].at[pl.ds(0, _B)], rows.at[slot], sem).wait()

        def pipeline(table, plane):
            pltpu.sync_copy(table.at[pl.ds(r0, _RPT)], acc.at[pl.ds(r0, _RPT)])
            pltpu.sync_copy(src2.at[pl.ds(row0, _CH)], srcb.at[0])
            pltpu.sync_copy(dst2.at[pl.ds(row0, _CH)], dstb.at[0])
            plsc.subcore_barrier()

            def fire_gather(idx_ref, slot):
                pltpu.async_copy(table.at[idx_ref], rows.at[slot], sem_g[slot])

            def process_chunk(cc, pc, first_chunk=False):
                for i in range(_CH):
                    q = i % _KR
                    wait_rows(q, sem_g[q])
                    pltpu.async_copy(
                        rows.at[q], acc.at[dstb.at[pc, i]], sem_s[q], add=True
                    )
                    if not (first_chunk and i == 0):
                        qp = (q + _KR - 1) % _KR
                        wait_rows(qp, sem_s[qp])
                    if i == 1:
                        nxt = row0 + (cc + 1) * _CH
                        pltpu.async_copy(src2.at[pl.ds(nxt, _CH)], srcb.at[1 - pc], sem_i)
                        pltpu.async_copy(dst2.at[pl.ds(nxt, _CH)], dstb.at[1 - pc], sem_i)
                    if i == _KR + 1:
                        pltpu.make_async_copy(src2.at[pl.ds(0, _CH)], srcb.at[1 - pc], sem_i).wait()
                        pltpu.make_async_copy(dst2.at[pl.ds(0, _CH)], dstb.at[1 - pc], sem_i).wait()
                    j = i + _KR - 1
                    if j < _CH:
                        fire_gather(srcb.at[pc, j], (q + _KR - 1) % _KR)
                    else:
                        fire_gather(srcb.at[1 - pc, j - _CH], (q + _KR - 1) % _KR)

            for b in range(_KR - 1):
                fire_gather(srcb.at[0, b], b)
            process_chunk(0, 0, first_chunk=True)
            process_chunk(1, 1)

            def outer(cp, carry):
                process_chunk(2 * cp, 0)
                process_chunk(2 * cp + 1, 1)
                return carry

            lax.fori_loop(1, _NCH // 2, outer, 0)
            for b in range(_KR - 1):
                wait_rows(b, sem_g[b])
            wait_rows(_KR - 1, sem_s[_KR - 1])
            plsc.subcore_barrier()
            pltpu.sync_copy(
                acc.at[pl.ds(r0, _RPT)], out.at[plane, pl.ds(r0, _RPT)]
            )

        @pl.when(c == 0)
        def _():
            for k in range(npairs):
                pipeline(tables[2 * k], 2 * k)

        @pl.when(c == 1)
        def _():
            for k in range(npairs):
                pipeline(tables[2 * k + 1], 2 * k + 1)

    return _sc_agg


_BM = 1024
_G = _NPAD // _BM


def _zblk(plane):
    return pl.BlockSpec((1, _BM, 64), lambda i, p=plane: (p, i, 0))


def _hblk():
    return pl.BlockSpec((_BM, 64), lambda i: (i, 0))


def _full(shp):
    return pl.BlockSpec(shp, lambda i: tuple(0 for _ in shp))


def _dot(a, b):
    return jnp.dot(a, b, preferred_element_type=jnp.float32)


def _mlp_single(z, w0, w1, ba, wb, bb, gs, be):
    def body(z0, z1, w0_, w1_, ba_, wb_, bb_, gs_, be_, o0, o1):
        a = jnp.maximum(_dot(z0[0], w0_[...]) + _dot(z1[0], w1_[...]) + ba_[...], 0.0)
        t = _dot(a, wb_[...]) + bb_[...]
        y = jnp.maximum(t * gs_[...] + be_[...], 0.0)
        o0[...] = y[:, :64]
        o1[...] = y[:, 64:]

    return pl.pallas_call(
        body,
        grid=(_G,),
        in_specs=[_zblk(0), _zblk(1),
                  _full(w0.shape), _full(w1.shape), _full(ba.shape),
                  _full(wb.shape), _full(bb.shape), _full(gs.shape), _full(be.shape)],
        out_specs=[_hblk(), _hblk()],
        out_shape=[jax.ShapeDtypeStruct((_NPAD, 64), jnp.float32)] * 2,
    )(z, z, w0, w1, ba, wb, bb, gs, be)


def _mlp_wide(z, w0, w1, ba, wb, bb, gs, be):
    def body(z0, z1, w0_, w1_, ba_, wb_, bb_, gs_, be_, o0, o1, o2, o3):
        a = jnp.maximum(_dot(z0[0], w0_[...]) + _dot(z1[0], w1_[...]) + ba_[...], 0.0)
        t = _dot(a, wb_[...]) + bb_[...]
        y = jnp.maximum(t * gs_[...] + be_[...], 0.0)
        o0[...] = y[:, :64]
        o1[...] = y[:, 64:128]
        o2[...] = y[:, 128:192]
        o3[...] = y[:, 192:]

    return pl.pallas_call(
        body,
        grid=(_G,),
        in_specs=[_zblk(0), _zblk(1),
                  _full(w0.shape), _full(w1.shape), _full(ba.shape),
                  _full(wb.shape), _full(bb.shape), _full(gs.shape), _full(be.shape)],
        out_specs=[_hblk()] * 4,
        out_shape=[jax.ShapeDtypeStruct((_NPAD, 64), jnp.float32)] * 4,
    )(z, z, w0, w1, ba, wb, bb, gs, be)


def _mlp_narrowing(z, ws, ba, wb, bb, gs, be):
    def body(z0, z1, z2, z3, w0_, w1_, w2_, w3_, ba_, wb_, bb_, gs_, be_, o0, o1):
        a = jnp.maximum(
            _dot(z0[0], w0_[...]) + _dot(z1[0], w1_[...])
            + _dot(z2[0], w2_[...]) + _dot(z3[0], w3_[...]) + ba_[...], 0.0)
        t = _dot(a, wb_[...]) + bb_[...]
        y = jnp.maximum(t * gs_[...] + be_[...], 0.0)
        o0[...] = y[:, :64]
        o1[...] = y[:, 64:]

    return pl.pallas_call(
        body,
        grid=(_G,),
        in_specs=[_zblk(0), _zblk(1), _zblk(2), _zblk(3)]
                 + [_full(w.shape) for w in ws]
                 + [_full(ba.shape), _full(wb.shape), _full(bb.shape),
                    _full(gs.shape), _full(be.shape)],
        out_specs=[_hblk()] * 2,
        out_shape=[jax.ShapeDtypeStruct((_NPAD, 64), jnp.float32)] * 2,
    )(z, z, z, z, *ws, ba, wb, bb, gs, be)


def _mlp_final(z, w0, w1, ba, wb, bb, gs, be, wc, bc):
    def body(z0, z1, w0_, w1_, ba_, wb_, bb_, gs_, be_, wc_, bc_, o):
        a = jnp.maximum(_dot(z0[0], w0_[...]) + _dot(z1[0], w1_[...]) + ba_[...], 0.0)
        t = _dot(a, wb_[...]) + bb_[...]
        y = jnp.maximum(t * gs_[...] + be_[...], 0.0)
        o[...] = _dot(y, wc_[...]) + bc_[...]

    return pl.pallas_call(
        body,
        grid=(_G,),
        in_specs=[_zblk(0), _zblk(1),
                  _full(w0.shape), _full(w1.shape), _full(ba.shape),
                  _full(wb.shape), _full(bb.shape), _full(gs.shape), _full(be.shape),
                  _full(wc.shape), _full(bc.shape)],
        out_specs=pl.BlockSpec((_BM, 128), lambda i: (i, 0)),
        out_shape=jax.ShapeDtypeStruct((_NPAD, 128), jnp.float32),
    )(z, z, w0, w1, ba, wb, bb, gs, be, wc, bc)


def kernel(x, edge_index, w1a, b1a, w1b, b1b, w2a, b2a, w2b, b2b,
           w3a, b3a, w3b, b3b, w4a, b4a, w4b, b4b,
           g1, be1, g2, be2, g3, be3, g4, be4, wc, bc):
    f32 = jnp.float32
    scale = 1.0 / jnp.sqrt(jnp.asarray(1.0 + _BN_EPS, f32))

    def row(v):
        return v.reshape(1, -1).astype(f32)

    gs1, gs2, gs3, gs4 = (row(g1) * scale, row(g2) * scale,
                          row(g3) * scale, row(g4) * scale)
    be1r, be2r, be3r, be4r = row(be1), row(be2), row(be3), row(be4)
    b1ar, b1br = row(b1a), row(b1b)
    b2ar, b2br = row(b2a), row(b2b)
    b3ar, b3br = row(b3a), row(b3b)
    b4ar, b4br = row(b4a), row(b4b)

    wcp = jnp.zeros((wc.shape[0], 128), f32).at[:, : wc.shape[1]].set(wc)
    bcp = jnp.zeros((1, 128), f32).at[0, : bc.shape[0]].set(bc)

    ei = edge_index.astype(jnp.int32)
    src, dst = ei[0], ei[1]
    npad_extra = _NPAD - _N
    ar = jnp.arange(_IROWS * _B - _E, dtype=jnp.int32)
    srcp = jnp.concatenate([src, _N + (ar % npad_extra)]).reshape(_IROWS, _B)
    dstp = jnp.concatenate([dst, _N + (ar % npad_extra)]).reshape(_IROWS, _B)

    xp = jnp.pad(x, ((0, npad_extra), (0, 0)))
    x_lo, x_hi = xp[:, :64], xp[:, 64:]

    w1a0, w1a1 = w1a[:64], w1a[64:]
    w2a0, w2a1 = w2a[:64], w2a[64:]
    w3s = (w3a[:64], w3a[64:128], w3a[128:192], w3a[192:])
    w4a0, w4a1 = w4a[:64], w4a[64:]

    sc_agg = _get_sc_agg()
    sc_agg2 = _get_sc_agg(2)
    z1 = sc_agg(x_lo, x_hi, srcp, dstp)
    h2lo, h2hi = _mlp_single(z1, w1a0, w1a1, b1ar, w1b, b1br, gs1, be1r)
    z2 = sc_agg(h2lo, h2hi, srcp, dstp)
    h3a, h3b, h3c, h3d = _mlp_wide(z2, w2a0, w2a1, b2ar, w2b, b2br, gs2, be2r)
    z3 = sc_agg2(h3a, h3b, h3c, h3d, srcp, dstp)
    h4lo, h4hi = _mlp_narrowing(z3, w3s, b3ar, w3b, b3br, gs3, be3r)
    z4 = sc_agg(h4lo, h4hi, srcp, dstp)
    out = _mlp_final(z4, w4a0, w4a1, b4ar, w4b, b4br, gs4, be4r, wcp, bcp)
    return out[:_N, : wc.shape[1]]


# speedup vs baseline: 1.0332x; 1.0332x over previous
import functools

import jax
import jax.numpy as jnp
from jax import lax
from jax.experimental import pallas as pl
from jax.experimental.pallas import tpu as pltpu
from jax.experimental.pallas import tpu_sc as plsc

_N = 10000
_NPAD = 10240
_E = 320000
_B = 128
_NC = 2
_NS = 16
_BT = 160
_CH = 16
_NCH = _BT // _CH
_KR = 8
_EPAD = _NS * _BT * _B
_IROWS = _NS * _BT + _CH
_RPT = _NPAD // _NS
_BN_EPS = 1e-5


@functools.cache
def _get_sc_agg(npairs=1):
    mesh = plsc.VectorSubcoreMesh(
        core_axis_name="c", subcore_axis_name="s", num_cores=_NC, num_subcores=_NS
    )

    @functools.partial(
        pl.kernel,
        out_type=jax.ShapeDtypeStruct((2 * npairs, _NPAD, 64), jnp.float32),
        mesh=mesh,
        compiler_params=pltpu.CompilerParams(use_tc_tiling_on_sc=False),
        scratch_types=[
            pltpu.VMEM((2, _CH, _B), jnp.int32),
            pltpu.VMEM((2, _CH, _B), jnp.int32),
            pltpu.VMEM((_KR, _B, 64), jnp.float32),
            pltpu.VMEM_SHARED((_NPAD, 64), jnp.float32),
            [pltpu.SemaphoreType.DMA] * _KR,
            [pltpu.SemaphoreType.DMA] * _KR,
            pltpu.SemaphoreType.DMA,
        ],
    )
    def _sc_agg(*refs):
        tables = refs[: 2 * npairs]
        src2, dst2, out = refs[2 * npairs: 2 * npairs + 3]
        srcb, dstb, rows, acc, sem_g, sem_s, sem_i = refs[2 * npairs + 3:]
        c = lax.axis_index("c")
        s = lax.axis_index("s")
        r0 = s * _RPT
        row0 = s * _BT

        def wait_rows(slot, sem):
            pltpu.make_async_copy(tables[0].at[pl.ds(0, _B)], rows.at[slot], sem).wait()

        def pipeline(table, plane):
            seed = pltpu.async_copy(
                table.at[pl.ds(r0, _RPT)], acc.at[pl.ds(r0, _RPT)], sem_i
            )
            pltpu.sync_copy(src2.at[pl.ds(row0, _CH)], srcb.at[0])
            pltpu.sync_copy(dst2.at[pl.ds(row0, _CH)], dstb.at[0])

            def fire_gather(idx_ref, slot):
                pltpu.async_copy(table.at[idx_ref], rows.at[slot], sem_g[slot])

            def process_chunk(cc, pc, first_chunk=False):
                for i in range(_CH):
                    q = i % _KR
                    wait_rows(q, sem_g[q])
                    pltpu.async_copy(
                        rows.at[q], acc.at[dstb.at[pc, i]], sem_s[q], add=True
                    )
                    if not (first_chunk and i == 0):
                        qp = (q + _KR - 1) % _KR
                        wait_rows(qp, sem_s[qp])
                    if i == 1:
                        nxt = row0 + (cc + 1) * _CH
                        pltpu.async_copy(src2.at[pl.ds(nxt, _CH)], srcb.at[1 - pc], sem_i)
                        pltpu.async_copy(dst2.at[pl.ds(nxt, _CH)], dstb.at[1 - pc], sem_i)
                    if i == _KR + 1:
                        pltpu.make_async_copy(src2.at[pl.ds(0, _CH)], srcb.at[1 - pc], sem_i).wait()
                        pltpu.make_async_copy(dst2.at[pl.ds(0, _CH)], dstb.at[1 - pc], sem_i).wait()
                    j = i + _KR - 1
                    if j < _CH:
                        fire_gather(srcb.at[pc, j], (q + _KR - 1) % _KR)
                    else:
                        fire_gather(srcb.at[1 - pc, j - _CH], (q + _KR - 1) % _KR)

            for b in range(_KR - 1):
                fire_gather(srcb.at[0, b], b)
            seed.wait()
            plsc.subcore_barrier()
            process_chunk(0, 0, first_chunk=True)
            process_chunk(1, 1)

            def outer(cp, carry):
                process_chunk(2 * cp, 0)
                process_chunk(2 * cp + 1, 1)
                return carry

            lax.fori_loop(1, _NCH // 2, outer, 0)
            for b in range(_KR - 1):
                wait_rows(b, sem_g[b])
            wait_rows(_KR - 1, sem_s[_KR - 1])
            plsc.subcore_barrier()
            pltpu.sync_copy(
                acc.at[pl.ds(r0, _RPT)], out.at[plane, pl.ds(r0, _RPT)]
            )

        @pl.when(c == 0)
        def _():
            for k in range(npairs):
                pipeline(tables[2 * k], 2 * k)

        @pl.when(c == 1)
        def _():
            for k in range(npairs):
                pipeline(tables[2 * k + 1], 2 * k + 1)

    return _sc_agg


_BM = 1024
_G = _NPAD // _BM


def _zblk(plane):
    return pl.BlockSpec((1, _BM, 64), lambda i, p=plane: (p, i, 0))


def _hblk():
    return pl.BlockSpec((_BM, 64), lambda i: (i, 0))


def _full(shp):
    return pl.BlockSpec(shp, lambda i: tuple(0 for _ in shp))


def _dot(a, b):
    return jnp.dot(a, b, preferred_element_type=jnp.float32)


def _mlp_single(z, w0, w1, ba, wb, bb, gs, be):
    def body(z0, z1, w0_, w1_, ba_, wb_, bb_, gs_, be_, o0, o1):
        a = jnp.maximum(_dot(z0[0], w0_[...]) + _dot(z1[0], w1_[...]) + ba_[...], 0.0)
        t = _dot(a, wb_[...]) + bb_[...]
        y = jnp.maximum(t * gs_[...] + be_[...], 0.0)
        o0[...] = y[:, :64]
        o1[...] = y[:, 64:]

    return pl.pallas_call(
        body,
        grid=(_G,),
        in_specs=[_zblk(0), _zblk(1),
                  _full(w0.shape), _full(w1.shape), _full(ba.shape),
                  _full(wb.shape), _full(bb.shape), _full(gs.shape), _full(be.shape)],
        out_specs=[_hblk(), _hblk()],
        out_shape=[jax.ShapeDtypeStruct((_NPAD, 64), jnp.float32)] * 2,
    )(z, z, w0, w1, ba, wb, bb, gs, be)


def _mlp_wide(z, w0, w1, ba, wb, bb, gs, be):
    def body(z0, z1, w0_, w1_, ba_, wb_, bb_, gs_, be_, o0, o1, o2, o3):
        a = jnp.maximum(_dot(z0[0], w0_[...]) + _dot(z1[0], w1_[...]) + ba_[...], 0.0)
        t = _dot(a, wb_[...]) + bb_[...]
        y = jnp.maximum(t * gs_[...] + be_[...], 0.0)
        o0[...] = y[:, :64]
        o1[...] = y[:, 64:128]
        o2[...] = y[:, 128:192]
        o3[...] = y[:, 192:]

    return pl.pallas_call(
        body,
        grid=(_G,),
        in_specs=[_zblk(0), _zblk(1),
                  _full(w0.shape), _full(w1.shape), _full(ba.shape),
                  _full(wb.shape), _full(bb.shape), _full(gs.shape), _full(be.shape)],
        out_specs=[_hblk()] * 4,
        out_shape=[jax.ShapeDtypeStruct((_NPAD, 64), jnp.float32)] * 4,
    )(z, z, w0, w1, ba, wb, bb, gs, be)


def _mlp_narrowing(za, zb, ws, ba, wb, bb, gs, be):
    def body(z0, z1, z2, z3, w0_, w1_, w2_, w3_, ba_, wb_, bb_, gs_, be_, o0, o1):
        a = jnp.maximum(
            _dot(z0[0], w0_[...]) + _dot(z1[0], w1_[...])
            + _dot(z2[0], w2_[...]) + _dot(z3[0], w3_[...]) + ba_[...], 0.0)
        t = _dot(a, wb_[...]) + bb_[...]
        y = jnp.maximum(t * gs_[...] + be_[...], 0.0)
        o0[...] = y[:, :64]
        o1[...] = y[:, 64:]

    return pl.pallas_call(
        body,
        grid=(_G,),
        in_specs=[_zblk(0), _zblk(1), _zblk(0), _zblk(1)]
                 + [_full(w.shape) for w in ws]
                 + [_full(ba.shape), _full(wb.shape), _full(bb.shape),
                    _full(gs.shape), _full(be.shape)],
        out_specs=[_hblk()] * 2,
        out_shape=[jax.ShapeDtypeStruct((_NPAD, 64), jnp.float32)] * 2,
    )(za, za, zb, zb, *ws, ba, wb, bb, gs, be)


def _mlp_final(z, w0, w1, ba, wb, bb, gs, be, wc, bc):
    def body(z0, z1, w0_, w1_, ba_, wb_, bb_, gs_, be_, wc_, bc_, o):
        a = jnp.maximum(_dot(z0[0], w0_[...]) + _dot(z1[0], w1_[...]) + ba_[...], 0.0)
        t = _dot(a, wb_[...]) + bb_[...]
        y = jnp.maximum(t * gs_[...] + be_[...], 0.0)
        o[...] = _dot(y, wc_[...]) + bc_[...]

    return pl.pallas_call(
        body,
        grid=(_G,),
        in_specs=[_zblk(0), _zblk(1),
                  _full(w0.shape), _full(w1.shape), _full(ba.shape),
                  _full(wb.shape), _full(bb.shape), _full(gs.shape), _full(be.shape),
                  _full(wc.shape), _full(bc.shape)],
        out_specs=pl.BlockSpec((_BM, 128), lambda i: (i, 0)),
        out_shape=jax.ShapeDtypeStruct((_NPAD, 128), jnp.float32),
    )(z, z, w0, w1, ba, wb, bb, gs, be, wc, bc)


def kernel(x, edge_index, w1a, b1a, w1b, b1b, w2a, b2a, w2b, b2b,
           w3a, b3a, w3b, b3b, w4a, b4a, w4b, b4b,
           g1, be1, g2, be2, g3, be3, g4, be4, wc, bc):
    f32 = jnp.float32
    scale = 1.0 / jnp.sqrt(jnp.asarray(1.0 + _BN_EPS, f32))

    def row(v):
        return v.reshape(1, -1).astype(f32)

    gs1, gs2, gs3, gs4 = (row(g1) * scale, row(g2) * scale,
                          row(g3) * scale, row(g4) * scale)
    be1r, be2r, be3r, be4r = row(be1), row(be2), row(be3), row(be4)
    b1ar, b1br = row(b1a), row(b1b)
    b2ar, b2br = row(b2a), row(b2b)
    b3ar, b3br = row(b3a), row(b3b)
    b4ar, b4br = row(b4a), row(b4b)

    wcp = jnp.zeros((wc.shape[0], 128), f32).at[:, : wc.shape[1]].set(wc)
    bcp = jnp.zeros((1, 128), f32).at[0, : bc.shape[0]].set(bc)

    ei = edge_index.astype(jnp.int32)
    src, dst = ei[0], ei[1]
    npad_extra = _NPAD - _N
    ar = jnp.arange(_IROWS * _B - _E, dtype=jnp.int32)
    srcp = jnp.concatenate([src, _N + (ar % npad_extra)]).reshape(_IROWS, _B)
    dstp = jnp.concatenate([dst, _N + (ar % npad_extra)]).reshape(_IROWS, _B)

    xp = jnp.pad(x, ((0, npad_extra), (0, 0)))
    x_lo, x_hi = xp[:, :64], xp[:, 64:]

    w1a0, w1a1 = w1a[:64], w1a[64:]
    w2a0, w2a1 = w2a[:64], w2a[64:]
    w3s = (w3a[:64], w3a[64:128], w3a[128:192], w3a[192:])
    w4a0, w4a1 = w4a[:64], w4a[64:]

    sc_agg = _get_sc_agg()
    z1 = sc_agg(x_lo, x_hi, srcp, dstp)
    h2lo, h2hi = _mlp_single(z1, w1a0, w1a1, b1ar, w1b, b1br, gs1, be1r)
    z2 = sc_agg(h2lo, h2hi, srcp, dstp)
    h3a, h3b, h3c, h3d = _mlp_wide(z2, w2a0, w2a1, b2ar, w2b, b2br, gs2, be2r)
    z3a = sc_agg(h3a, h3b, srcp, dstp)
    z3b = sc_agg(h3c, h3d, srcp, dstp)
    h4lo, h4hi = _mlp_narrowing(z3a, z3b, w3s, b3ar, w3b, b3br, gs3, be3r)
    z4 = sc_agg(h4lo, h4hi, srcp, dstp)
    out = _mlp_final(z4, w4a0, w4a1, b4ar, w4b, b4br, gs4, be4r, wcp, bcp)
    return out[:_N, : wc.shape[1]]


# speedup vs baseline: 1.0338x; 1.0005x over previous
import functools

import jax
import jax.numpy as jnp
from jax import lax
from jax.experimental import pallas as pl
from jax.experimental.pallas import tpu as pltpu
from jax.experimental.pallas import tpu_sc as plsc

_N = 10000
_NPAD = 10240
_E = 320000
_B = 128
_NC = 2
_NS = 16
_BT = 160
_CH = 16
_NCH = _BT // _CH
_KR = 8
_EPAD = _NS * _BT * _B
_IROWS = _NS * _BT + _CH
_RPT = _NPAD // _NS
_BN_EPS = 1e-5


@functools.cache
def _get_sc_agg(npairs=1):
    mesh = plsc.VectorSubcoreMesh(
        core_axis_name="c", subcore_axis_name="s", num_cores=_NC, num_subcores=_NS
    )

    @functools.partial(
        pl.kernel,
        out_type=jax.ShapeDtypeStruct((2 * npairs, _NPAD, 64), jnp.float32),
        mesh=mesh,
        compiler_params=pltpu.CompilerParams(
            use_tc_tiling_on_sc=False, skip_device_barrier=True
        ),
        scratch_types=[
            pltpu.VMEM((2, _CH, _B), jnp.int32),
            pltpu.VMEM((2, _CH, _B), jnp.int32),
            pltpu.VMEM((_KR, _B, 64), jnp.float32),
            pltpu.VMEM_SHARED((_NPAD, 64), jnp.float32),
            [pltpu.SemaphoreType.DMA] * _KR,
            [pltpu.SemaphoreType.DMA] * _KR,
            pltpu.SemaphoreType.DMA,
        ],
    )
    def _sc_agg(*refs):
        tables = refs[: 2 * npairs]
        src2, dst2, out = refs[2 * npairs: 2 * npairs + 3]
        srcb, dstb, rows, acc, sem_g, sem_s, sem_i = refs[2 * npairs + 3:]
        c = lax.axis_index("c")
        s = lax.axis_index("s")
        r0 = s * _RPT
        row0 = s * _BT

        def wait_rows(slot, sem):
            pltpu.make_async_copy(tables[0].at[pl.ds(0, _B)], rows.at[slot], sem).wait()

        def pipeline(table, plane):
            seed = pltpu.async_copy(
                table.at[pl.ds(r0, _RPT)], acc.at[pl.ds(r0, _RPT)], sem_i
            )
            pltpu.sync_copy(src2.at[pl.ds(row0, _CH)], srcb.at[0])
            pltpu.sync_copy(dst2.at[pl.ds(row0, _CH)], dstb.at[0])

            def fire_gather(idx_ref, slot):
                pltpu.async_copy(table.at[idx_ref], rows.at[slot], sem_g[slot])

            def process_chunk(cc, pc, first_chunk=False):
                for i in range(_CH):
                    q = i % _KR
                    wait_rows(q, sem_g[q])
                    pltpu.async_copy(
                        rows.at[q], acc.at[dstb.at[pc, i]], sem_s[q], add=True
                    )
                    if not (first_chunk and i == 0):
                        qp = (q + _KR - 1) % _KR
                        wait_rows(qp, sem_s[qp])
                    if i == 1:
                        nxt = row0 + (cc + 1) * _CH
                        pltpu.async_copy(src2.at[pl.ds(nxt, _CH)], srcb.at[1 - pc], sem_i)
                        pltpu.async_copy(dst2.at[pl.ds(nxt, _CH)], dstb.at[1 - pc], sem_i)
                    if i == _KR + 1:
                        pltpu.make_async_copy(src2.at[pl.ds(0, _CH)], srcb.at[1 - pc], sem_i).wait()
                        pltpu.make_async_copy(dst2.at[pl.ds(0, _CH)], dstb.at[1 - pc], sem_i).wait()
                    j = i + _KR - 1
                    if j < _CH:
                        fire_gather(srcb.at[pc, j], (q + _KR - 1) % _KR)
                    else:
                        fire_gather(srcb.at[1 - pc, j - _CH], (q + _KR - 1) % _KR)

            for b in range(_KR - 1):
                fire_gather(srcb.at[0, b], b)
            seed.wait()
            plsc.subcore_barrier()
            process_chunk(0, 0, first_chunk=True)
            process_chunk(1, 1)

            def outer(cp, carry):
                process_chunk(2 * cp, 0)
                process_chunk(2 * cp + 1, 1)
                return carry

            lax.fori_loop(1, _NCH // 2, outer, 0)
            for b in range(_KR - 1):
                wait_rows(b, sem_g[b])
            wait_rows(_KR - 1, sem_s[_KR - 1])
            plsc.subcore_barrier()
            pltpu.sync_copy(
                acc.at[pl.ds(r0, _RPT)], out.at[plane, pl.ds(r0, _RPT)]
            )

        @pl.when(c == 0)
        def _():
            for k in range(npairs):
                pipeline(tables[2 * k], 2 * k)

        @pl.when(c == 1)
        def _():
            for k in range(npairs):
                pipeline(tables[2 * k + 1], 2 * k + 1)

    return _sc_agg


_BM = 1024
_G = _NPAD // _BM


def _zblk(plane):
    return pl.BlockSpec((1, _BM, 64), lambda i, p=plane: (p, i, 0))


def _hblk():
    return pl.BlockSpec((_BM, 64), lambda i: (i, 0))


def _full(shp):
    return pl.BlockSpec(shp, lambda i: tuple(0 for _ in shp))


def _dot(a, b):
    return jnp.dot(a, b, preferred_element_type=jnp.float32)


def _mlp_single(z, w0, w1, ba, wb, bb, gs, be):
    def body(z0, z1, w0_, w1_, ba_, wb_, bb_, gs_, be_, o0, o1):
        a = jnp.maximum(_dot(z0[0], w0_[...]) + _dot(z1[0], w1_[...]) + ba_[...], 0.0)
        t = _dot(a, wb_[...]) + bb_[...]
        y = jnp.maximum(t * gs_[...] + be_[...], 0.0)
        o0[...] = y[:, :64]
        o1[...] = y[:, 64:]

    return pl.pallas_call(
        body,
        grid=(_G,),
        in_specs=[_zblk(0), _zblk(1),
                  _full(w0.shape), _full(w1.shape), _full(ba.shape),
                  _full(wb.shape), _full(bb.shape), _full(gs.shape), _full(be.shape)],
        out_specs=[_hblk(), _hblk()],
        out_shape=[jax.ShapeDtypeStruct((_NPAD, 64), jnp.float32)] * 2,
    )(z, z, w0, w1, ba, wb, bb, gs, be)


def _mlp_wide(z, w0, w1, ba, wb, bb, gs, be):
    def body(z0, z1, w0_, w1_, ba_, wb_, bb_, gs_, be_, o0, o1, o2, o3):
        a = jnp.maximum(_dot(z0[0], w0_[...]) + _dot(z1[0], w1_[...]) + ba_[...], 0.0)
        t = _dot(a, wb_[...]) + bb_[...]
        y = jnp.maximum(t * gs_[...] + be_[...], 0.0)
        o0[...] = y[:, :64]
        o1[...] = y[:, 64:128]
        o2[...] = y[:, 128:192]
        o3[...] = y[:, 192:]

    return pl.pallas_call(
        body,
        grid=(_G,),
        in_specs=[_zblk(0), _zblk(1),
                  _full(w0.shape), _full(w1.shape), _full(ba.shape),
                  _full(wb.shape), _full(bb.shape), _full(gs.shape), _full(be.shape)],
        out_specs=[_hblk()] * 4,
        out_shape=[jax.ShapeDtypeStruct((_NPAD, 64), jnp.float32)] * 4,
    )(z, z, w0, w1, ba, wb, bb, gs, be)


def _mlp_narrowing(za, zb, ws, ba, wb, bb, gs, be):
    def body(z0, z1, z2, z3, w0_, w1_, w2_, w3_, ba_, wb_, bb_, gs_, be_, o0, o1):
        a = jnp.maximum(
            _dot(z0[0], w0_[...]) + _dot(z1[0], w1_[...])
            + _dot(z2[0], w2_[...]) + _dot(z3[0], w3_[...]) + ba_[...], 0.0)
        t = _dot(a, wb_[...]) + bb_[...]
        y = jnp.maximum(t * gs_[...] + be_[...], 0.0)
        o0[...] = y[:, :64]
        o1[...] = y[:, 64:]

    return pl.pallas_call(
        body,
        grid=(_G,),
        in_specs=[_zblk(0), _zblk(1), _zblk(0), _zblk(1)]
                 + [_full(w.shape) for w in ws]
                 + [_full(ba.shape), _full(wb.shape), _full(bb.shape),
                    _full(gs.shape), _full(be.shape)],
        out_specs=[_hblk()] * 2,
        out_shape=[jax.ShapeDtypeStruct((_NPAD, 64), jnp.float32)] * 2,
    )(za, za, zb, zb, *ws, ba, wb, bb, gs, be)


def _mlp_final(z, w0, w1, ba, wb, bb, gs, be, wc, bc):
    def body(z0, z1, w0_, w1_, ba_, wb_, bb_, gs_, be_, wc_, bc_, o):
        a = jnp.maximum(_dot(z0[0], w0_[...]) + _dot(z1[0], w1_[...]) + ba_[...], 0.0)
        t = _dot(a, wb_[...]) + bb_[...]
        y = jnp.maximum(t * gs_[...] + be_[...], 0.0)
        o[...] = _dot(y, wc_[...]) + bc_[...]

    return pl.pallas_call(
        body,
        grid=(_G,),
        in_specs=[_zblk(0), _zblk(1),
                  _full(w0.shape), _full(w1.shape), _full(ba.shape),
                  _full(wb.shape), _full(bb.shape), _full(gs.shape), _full(be.shape),
                  _full(wc.shape), _full(bc.shape)],
        out_specs=pl.BlockSpec((_BM, 128), lambda i: (i, 0)),
        out_shape=jax.ShapeDtypeStruct((_NPAD, 128), jnp.float32),
    )(z, z, w0, w1, ba, wb, bb, gs, be, wc, bc)


def kernel(x, edge_index, w1a, b1a, w1b, b1b, w2a, b2a, w2b, b2b,
           w3a, b3a, w3b, b3b, w4a, b4a, w4b, b4b,
           g1, be1, g2, be2, g3, be3, g4, be4, wc, bc):
    f32 = jnp.float32
    scale = 1.0 / jnp.sqrt(jnp.asarray(1.0 + _BN_EPS, f32))

    def row(v):
        return v.reshape(1, -1).astype(f32)

    gs1, gs2, gs3, gs4 = (row(g1) * scale, row(g2) * scale,
                          row(g3) * scale, row(g4) * scale)
    be1r, be2r, be3r, be4r = row(be1), row(be2), row(be3), row(be4)
    b1ar, b1br = row(b1a), row(b1b)
    b2ar, b2br = row(b2a), row(b2b)
    b3ar, b3br = row(b3a), row(b3b)
    b4ar, b4br = row(b4a), row(b4b)

    wcp = jnp.zeros((wc.shape[0], 128), f32).at[:, : wc.shape[1]].set(wc)
    bcp = jnp.zeros((1, 128), f32).at[0, : bc.shape[0]].set(bc)

    ei = edge_index.astype(jnp.int32)
    src, dst = ei[0], ei[1]
    npad_extra = _NPAD - _N
    ar = jnp.arange(_IROWS * _B - _E, dtype=jnp.int32)
    srcp = jnp.concatenate([src, _N + (ar % npad_extra)]).reshape(_IROWS, _B)
    dstp = jnp.concatenate([dst, _N + (ar % npad_extra)]).reshape(_IROWS, _B)

    xp = jnp.pad(x, ((0, npad_extra), (0, 0)))
    x_lo, x_hi = xp[:, :64], xp[:, 64:]

    w1a0, w1a1 = w1a[:64], w1a[64:]
    w2a0, w2a1 = w2a[:64], w2a[64:]
    w3s = (w3a[:64], w3a[64:128], w3a[128:192], w3a[192:])
    w4a0, w4a1 = w4a[:64], w4a[64:]

    sc_agg = _get_sc_agg()
    z1 = sc_agg(x_lo, x_hi, srcp, dstp)
    h2lo, h2hi = _mlp_single(z1, w1a0, w1a1, b1ar, w1b, b1br, gs1, be1r)
    z2 = sc_agg(h2lo, h2hi, srcp, dstp)
    h3a, h3b, h3c, h3d = _mlp_wide(z2, w2a0, w2a1, b2ar, w2b, b2br, gs2, be2r)
    z3a = sc_agg(h3a, h3b, srcp, dstp)
    z3b = sc_agg(h3c, h3d, srcp, dstp)
    h4lo, h4hi = _mlp_narrowing(z3a, z3b, w3s, b3ar, w3b, b3br, gs3, be3r)
    z4 = sc_agg(h4lo, h4hi, srcp, dstp)
    out = _mlp_final(z4, w4a0, w4a1, b4ar, w4b, b4br, gs4, be4r, wcp, bcp)
    return out[:_N, : wc.shape[1]]


# speedup vs baseline: 1.0360x; 1.0022x over previous
import functools

import jax
import jax.numpy as jnp
from jax import lax
from jax.experimental import pallas as pl
from jax.experimental.pallas import tpu as pltpu
from jax.experimental.pallas import tpu_sc as plsc

_N = 10000
_NPAD = 10240
_E = 320000
_B = 128
_NC = 2
_NS = 16
_BT = 160
_CH = 16
_NCH = _BT // _CH
_KR = 8
_EPAD = _NS * _BT * _B
_IROWS = _NS * _BT + _CH
_RPT = _NPAD // _NS
_BN_EPS = 1e-5


@functools.cache
def _get_sc_agg(npairs=1):
    mesh = plsc.VectorSubcoreMesh(
        core_axis_name="c", subcore_axis_name="s", num_cores=_NC, num_subcores=_NS
    )

    @functools.partial(
        pl.kernel,
        out_type=jax.ShapeDtypeStruct((2 * npairs, _NPAD, 64), jnp.float32),
        mesh=mesh,
        compiler_params=pltpu.CompilerParams(use_tc_tiling_on_sc=False),
        scratch_types=[
            pltpu.VMEM((2, _CH, _B), jnp.int32),
            pltpu.VMEM((2, _CH, _B), jnp.int32),
            pltpu.VMEM((_KR, _B, 64), jnp.float32),
            pltpu.VMEM_SHARED((_NPAD, 64), jnp.float32),
            [pltpu.SemaphoreType.DMA] * _KR,
            [pltpu.SemaphoreType.DMA] * _KR,
            pltpu.SemaphoreType.DMA,
        ],
    )
    def _sc_agg(*refs):
        tables = refs[: 2 * npairs]
        src2, dst2, out = refs[2 * npairs: 2 * npairs + 3]
        srcb, dstb, rows, acc, sem_g, sem_s, sem_i = refs[2 * npairs + 3:]
        c = lax.axis_index("c")
        s = lax.axis_index("s")
        r0 = s * _RPT
        row0 = s * _BT

        def wait_rows(slot, sem):
            pltpu.make_async_copy(tables[0].at[pl.ds(0, _B)], rows.at[slot], sem).wait()

        def pipeline(table, plane):
            seed = pltpu.async_copy(
                table.at[pl.ds(r0, _RPT)], acc.at[pl.ds(r0, _RPT)], sem_s[0]
            )
            i0 = pltpu.async_copy(src2.at[pl.ds(row0, _CH)], srcb.at[0], sem_i)
            i1 = pltpu.async_copy(dst2.at[pl.ds(row0, _CH)], dstb.at[0], sem_i)
            i0.wait()
            i1.wait()

            def fire_gather(idx_ref, slot):
                pltpu.async_copy(table.at[idx_ref], rows.at[slot], sem_g[slot])

            def process_chunk(cc, pc, first_chunk=False):
                for i in range(_CH):
                    q = i % _KR
                    wait_rows(q, sem_g[q])
                    pltpu.async_copy(
                        rows.at[q], acc.at[dstb.at[pc, i]], sem_s[q], add=True
                    )
                    if not (first_chunk and i == 0):
                        qp = (q + _KR - 1) % _KR
                        wait_rows(qp, sem_s[qp])
                    if i == 1:
                        nxt = row0 + (cc + 1) * _CH
                        pltpu.async_copy(src2.at[pl.ds(nxt, _CH)], srcb.at[1 - pc], sem_i)
                        pltpu.async_copy(dst2.at[pl.ds(nxt, _CH)], dstb.at[1 - pc], sem_i)
                    if i == _KR + 1:
                        pltpu.make_async_copy(src2.at[pl.ds(0, _CH)], srcb.at[1 - pc], sem_i).wait()
                        pltpu.make_async_copy(dst2.at[pl.ds(0, _CH)], dstb.at[1 - pc], sem_i).wait()
                    j = i + _KR - 1
                    if j < _CH:
                        fire_gather(srcb.at[pc, j], (q + _KR - 1) % _KR)
                    else:
                        fire_gather(srcb.at[1 - pc, j - _CH], (q + _KR - 1) % _KR)

            for b in range(_KR - 1):
                fire_gather(srcb.at[0, b], b)
            seed.wait()
            plsc.subcore_barrier()
            process_chunk(0, 0, first_chunk=True)
            process_chunk(1, 1)

            def outer(cp, carry):
                process_chunk(2 * cp, 0)
                process_chunk(2 * cp + 1, 1)
                return carry

            lax.fori_loop(1, _NCH // 2, outer, 0)
            for b in range(_KR - 1):
                wait_rows(b, sem_g[b])
            wait_rows(_KR - 1, sem_s[_KR - 1])
            plsc.subcore_barrier()
            pltpu.sync_copy(
                acc.at[pl.ds(r0, _RPT)], out.at[plane, pl.ds(r0, _RPT)]
            )

        @pl.when(c == 0)
        def _():
            for k in range(npairs):
                pipeline(tables[2 * k], 2 * k)

        @pl.when(c == 1)
        def _():
            for k in range(npairs):
                pipeline(tables[2 * k + 1], 2 * k + 1)

    return _sc_agg


_BM = 1024
_G = _NPAD // _BM


def _zblk(plane):
    return pl.BlockSpec((1, _BM, 64), lambda i, p=plane: (p, i, 0))


def _hblk():
    return pl.BlockSpec((_BM, 64), lambda i: (i, 0))


def _full(shp):
    return pl.BlockSpec(shp, lambda i: tuple(0 for _ in shp))


def _dot(a, b):
    return jnp.dot(a, b, preferred_element_type=jnp.float32)


def _mlp_single(z, w0, w1, ba, wb, bb, gs, be):
    def body(z0, z1, w0_, w1_, ba_, wb_, bb_, gs_, be_, o0, o1):
        a = jnp.maximum(_dot(z0[0], w0_[...]) + _dot(z1[0], w1_[...]) + ba_[...], 0.0)
        t = _dot(a, wb_[...]) + bb_[...]
        y = jnp.maximum(t * gs_[...] + be_[...], 0.0)
        o0[...] = y[:, :64]
        o1[...] = y[:, 64:]

    return pl.pallas_call(
        body,
        grid=(_G,),
        in_specs=[_zblk(0), _zblk(1),
                  _full(w0.shape), _full(w1.shape), _full(ba.shape),
                  _full(wb.shape), _full(bb.shape), _full(gs.shape), _full(be.shape)],
        out_specs=[_hblk(), _hblk()],
        out_shape=[jax.ShapeDtypeStruct((_NPAD, 64), jnp.float32)] * 2,
    )(z, z, w0, w1, ba, wb, bb, gs, be)


def _mlp_wide(z, w0, w1, ba, wb, bb, gs, be):
    def body(z0, z1, w0_, w1_, ba_, wb_, bb_, gs_, be_, o0, o1, o2, o3):
        a = jnp.maximum(_dot(z0[0], w0_[...]) + _dot(z1[0], w1_[...]) + ba_[...], 0.0)
        t = _dot(a, wb_[...]) + bb_[...]
        y = jnp.maximum(t * gs_[...] + be_[...], 0.0)
        o0[...] = y[:, :64]
        o1[...] = y[:, 64:128]
        o2[...] = y[:, 128:192]
        o3[...] = y[:, 192:]

    return pl.pallas_call(
        body,
        grid=(_G,),
        in_specs=[_zblk(0), _zblk(1),
                  _full(w0.shape), _full(w1.shape), _full(ba.shape),
                  _full(wb.shape), _full(bb.shape), _full(gs.shape), _full(be.shape)],
        out_specs=[_hblk()] * 4,
        out_shape=[jax.ShapeDtypeStruct((_NPAD, 64), jnp.float32)] * 4,
    )(z, z, w0, w1, ba, wb, bb, gs, be)


def _mlp_narrowing(za, zb, ws, ba, wb, bb, gs, be):
    def body(z0, z1, z2, z3, w0_, w1_, w2_, w3_, ba_, wb_, bb_, gs_, be_, o0, o1):
        a = jnp.maximum(
            _dot(z0[0], w0_[...]) + _dot(z1[0], w1_[...])
            + _dot(z2[0], w2_[...]) + _dot(z3[0], w3_[...]) + ba_[...], 0.0)
        t = _dot(a, wb_[...]) + bb_[...]
        y = jnp.maximum(t * gs_[...] + be_[...], 0.0)
        o0[...] = y[:, :64]
        o1[...] = y[:, 64:]

    return pl.pallas_call(
        body,
        grid=(_G,),
        in_specs=[_zblk(0), _zblk(1), _zblk(0), _zblk(1)]
                 + [_full(w.shape) for w in ws]
                 + [_full(ba.shape), _full(wb.shape), _full(bb.shape),
                    _full(gs.shape), _full(be.shape)],
        out_specs=[_hblk()] * 2,
        out_shape=[jax.ShapeDtypeStruct((_NPAD, 64), jnp.float32)] * 2,
    )(za, za, zb, zb, *ws, ba, wb, bb, gs, be)


def _mlp_final(z, w0, w1, ba, wb, bb, gs, be, wc, bc):
    def body(z0, z1, w0_, w1_, ba_, wb_, bb_, gs_, be_, wc_, bc_, o):
        a = jnp.maximum(_dot(z0[0], w0_[...]) + _dot(z1[0], w1_[...]) + ba_[...], 0.0)
        t = _dot(a, wb_[...]) + bb_[...]
        y = jnp.maximum(t * gs_[...] + be_[...], 0.0)
        o[...] = _dot(y, wc_[...]) + bc_[...]

    return pl.pallas_call(
        body,
        grid=(_G,),
        in_specs=[_zblk(0), _zblk(1),
                  _full(w0.shape), _full(w1.shape), _full(ba.shape),
                  _full(wb.shape), _full(bb.shape), _full(gs.shape), _full(be.shape),
                  _full(wc.shape), _full(bc.shape)],
        out_specs=pl.BlockSpec((_BM, 16), lambda i: (i, 0)),
        out_shape=jax.ShapeDtypeStruct((_N, 16), jnp.float32),
    )(z, z, w0, w1, ba, wb, bb, gs, be, wc, bc)


def kernel(x, edge_index, w1a, b1a, w1b, b1b, w2a, b2a, w2b, b2b,
           w3a, b3a, w3b, b3b, w4a, b4a, w4b, b4b,
           g1, be1, g2, be2, g3, be3, g4, be4, wc, bc):
    f32 = jnp.float32
    scale = 1.0 / jnp.sqrt(jnp.asarray(1.0 + _BN_EPS, f32))

    def row(v):
        return v.reshape(1, -1).astype(f32)

    gs1, gs2, gs3, gs4 = (row(g1) * scale, row(g2) * scale,
                          row(g3) * scale, row(g4) * scale)
    be1r, be2r, be3r, be4r = row(be1), row(be2), row(be3), row(be4)
    b1ar, b1br = row(b1a), row(b1b)
    b2ar, b2br = row(b2a), row(b2b)
    b3ar, b3br = row(b3a), row(b3b)
    b4ar, b4br = row(b4a), row(b4b)

    wcp = wc.astype(f32)
    bcp = row(bc)

    ei = edge_index.astype(jnp.int32)
    src, dst = ei[0], ei[1]
    npad_extra = _NPAD - _N
    ar = jnp.arange(_IROWS * _B - _E, dtype=jnp.int32)
    srcp = jnp.concatenate([src, _N + (ar % npad_extra)]).reshape(_IROWS, _B)
    dstp = jnp.concatenate([dst, _N + (ar % npad_extra)]).reshape(_IROWS, _B)

    xp = jnp.pad(x, ((0, npad_extra), (0, 0)))
    x_lo, x_hi = xp[:, :64], xp[:, 64:]

    w1a0, w1a1 = w1a[:64], w1a[64:]
    w2a0, w2a1 = w2a[:64], w2a[64:]
    w3s = (w3a[:64], w3a[64:128], w3a[128:192], w3a[192:])
    w4a0, w4a1 = w4a[:64], w4a[64:]

    sc_agg = _get_sc_agg()
    z1 = sc_agg(x_lo, x_hi, srcp, dstp)
    h2lo, h2hi = _mlp_single(z1, w1a0, w1a1, b1ar, w1b, b1br, gs1, be1r)
    z2 = sc_agg(h2lo, h2hi, srcp, dstp)
    h3a, h3b, h3c, h3d = _mlp_wide(z2, w2a0, w2a1, b2ar, w2b, b2br, gs2, be2r)
    z3a = sc_agg(h3a, h3b, srcp, dstp)
    z3b = sc_agg(h3c, h3d, srcp, dstp)
    h4lo, h4hi = _mlp_narrowing(z3a, z3b, w3s, b3ar, w3b, b3br, gs3, be3r)
    z4 = sc_agg(h4lo, h4hi, srcp, dstp)
    return _mlp_final(z4, w4a0, w4a1, b4ar, w4b, b4br, gs4, be4r, wcp, bcp)
